# baseline (speedup 1.0000x reference)
import jax
import jax.numpy as jnp
from jax import lax
from jax.experimental import pallas as pl
from jax.experimental.pallas import tpu as pltpu

B = 8
NMAX = 512
H = 16
F = 768
N_TOTAL = 2048
NEG = -1000000000.0
CHUNK = 64
WIN = CHUNK + 8


def _kernel(ids_ref, x_hbm, dense_hbm, mask_hbm, xv, tiles, dsc,
            semx, semm, semd):
    pltpu.make_async_copy(x_hbm, xv, semx).start()

    ids = ids_ref[...]
    cnts = [jnp.sum((ids == b).astype(jnp.int32)) for b in range(B)]
    cbs = [jnp.sum((ids < b).astype(jnp.int32)) for b in range(B)]

    col = lax.broadcasted_iota(jnp.int32, (NMAX, NMAX), 1)
    for b in range(B):
        tiles[b] = jnp.where(col >= cnts[b], NEG, 0.0)
        for h in range(H):
            for q in range(2):
                pltpu.make_async_copy(
                    tiles.at[b, pl.ds(q * 256, 256)],
                    mask_hbm.at[b, h, pl.ds(q * 256, 256)],
                    semm,
                ).start()

    pltpu.make_async_copy(x_hbm, xv, semx).wait()
    kio = lax.broadcasted_iota(jnp.int32, (CHUNK, 1), 0)
    for b in range(B):
        for j in range(NMAX // CHUNK):
            start = cbs[b] + j * CHUNK
            s = jnp.minimum((start // 8) * 8, N_TOTAL - WIN)
            d = start - s
            win = xv[pl.ds(s, WIN), :]
            rolled = pltpu.roll(win, (WIN - d) % WIN, axis=0)[:CHUNK, :]
            dsc[b, pl.ds(j * CHUNK, CHUNK), :] = jnp.where(
                kio + j * CHUNK < cnts[b], rolled, 0.0
            )
        pltpu.make_async_copy(dsc.at[b], dense_hbm.at[b], semd).start()

    for b in range(B):
        pltpu.make_async_copy(dsc.at[b], dense_hbm.at[b], semd).wait()
    for b in range(B):
        for h in range(H):
            for q in range(2):
                pltpu.make_async_copy(
                    tiles.at[b, pl.ds(q * 256, 256)],
                    mask_hbm.at[b, h, pl.ds(q * 256, 256)],
                    semm,
                ).wait()


def kernel(x, batch_ids):
    ids2d = batch_ids.astype(jnp.int32).reshape(16, 128)
    dense_x, attn_mask = pl.pallas_call(
        _kernel,
        in_specs=[
            pl.BlockSpec((16, 128), lambda: (0, 0)),
            pl.BlockSpec(memory_space=pl.ANY),
        ],
        out_specs=[
            pl.BlockSpec(memory_space=pl.ANY),
            pl.BlockSpec(memory_space=pl.ANY),
        ],
        out_shape=[
            jax.ShapeDtypeStruct((B, NMAX, F), x.dtype),
            jax.ShapeDtypeStruct((B, H, NMAX, NMAX), jnp.float32),
        ],
        scratch_shapes=[
            pltpu.VMEM((N_TOTAL, F), jnp.float32),
            pltpu.VMEM((B, NMAX, NMAX), jnp.float32),
            pltpu.VMEM((B, NMAX, F), jnp.float32),
            pltpu.SemaphoreType.DMA,
            pltpu.SemaphoreType.DMA,
            pltpu.SemaphoreType.DMA,
        ],
    )(ids2d, x)
    return dense_x, attn_mask
